# baseline (speedup 1.0000x reference)
import jax
import jax.numpy as jnp
from jax.experimental import pallas as pl
from jax.experimental.pallas import tpu as pltpu

BLOCK_S = 1024


def _body(scale_ref, x_ref, pos_ref, out_ref):
    out_ref[...] = x_ref[...] + pos_ref[...] * scale_ref[0]


def kernel(x, pos_embed, scale):
    B, S, D = x.shape
    num_s = S // BLOCK_S

    grid_spec = pltpu.PrefetchScalarGridSpec(
        num_scalar_prefetch=1,
        grid=(num_s,),
        in_specs=[
            pl.BlockSpec((B, BLOCK_S, D), lambda s, *_: (0, s, 0)),
            pl.BlockSpec((1, BLOCK_S, D), lambda s, *_: (0, s, 0)),
        ],
        out_specs=pl.BlockSpec((B, BLOCK_S, D), lambda s, *_: (0, s, 0)),
    )

    return pl.pallas_call(
        _body,
        grid_spec=grid_spec,
        out_shape=jax.ShapeDtypeStruct((B, S, D), x.dtype),
        compiler_params=pltpu.CompilerParams(
            dimension_semantics=("arbitrary",),
        ),
    )(scale, x, pos_embed[None, :S])
